# Initial kernel scaffold; baseline (speedup 1.0000x reference)
import jax
import jax.numpy as jnp
from jax.experimental import pallas as pl


def kernel(input_distances, latent_distances):
    raise NotImplementedError("write your pallas kernel here")


# speedup vs baseline: 41.1105x; 41.1105x over previous
import jax
import jax.numpy as jnp
from jax.experimental import pallas as pl
from jax.experimental.pallas import tpu as pltpu

_N = 2048
_S = 16
_L = 128
_BIG = 3.0e38


def _mst_loss_kernel(a_ref, b_ref, out_ref):
    iota = jax.lax.broadcasted_iota(jnp.int32, (_S, _L), 0) * _L + \
        jax.lax.broadcasted_iota(jnp.int32, (_S, _L), 1)
    mask0 = jnp.where(iota == 0, jnp.float32(_BIG), jnp.float32(0.0))

    dist_a = a_ref[0]
    cross_a = b_ref[0]
    dist_b = b_ref[0]
    cross_b = a_ref[0]

    def step(_, carry):
        dist_a, cross_a, mask_a, loss_a, dist_b, cross_b, mask_b, loss_b = carry

        masked_a = dist_a + mask_a
        masked_b = dist_b + mask_b
        m_a = jnp.min(masked_a)
        m_b = jnp.min(masked_b)
        j_a = jnp.min(jnp.where(masked_a == m_a, iota, _N))
        j_b = jnp.min(jnp.where(masked_b == m_b, iota, _N))
        sel_a = iota == j_a
        sel_b = iota == j_b
        cv_a = jnp.sum(jnp.where(sel_a, cross_a, jnp.float32(0.0)))
        cv_b = jnp.sum(jnp.where(sel_b, cross_b, jnp.float32(0.0)))
        loss_a = loss_a + (m_a - cv_a) ** 2
        loss_b = loss_b + (m_b - cv_b) ** 2
        mask_a = jnp.where(sel_a, jnp.float32(_BIG), mask_a)
        mask_b = jnp.where(sel_b, jnp.float32(_BIG), mask_b)

        row_aa = a_ref[j_a]
        row_ab = b_ref[j_a]
        row_bb = b_ref[j_b]
        row_ba = a_ref[j_b]
        better_a = row_aa < dist_a
        better_b = row_bb < dist_b
        dist_a = jnp.where(better_a, row_aa, dist_a)
        cross_a = jnp.where(better_a, row_ab, cross_a)
        dist_b = jnp.where(better_b, row_bb, dist_b)
        cross_b = jnp.where(better_b, row_ba, cross_b)
        return (dist_a, cross_a, mask_a, loss_a,
                dist_b, cross_b, mask_b, loss_b)

    carry = (dist_a, cross_a, mask0, jnp.float32(0.0),
             dist_b, cross_b, mask0, jnp.float32(0.0))
    carry = jax.lax.fori_loop(0, _N - 1, step, carry)
    out_ref[0, 0] = carry[3] + carry[7]


def kernel(input_distances, latent_distances):
    a = input_distances.reshape(_N, _S, _L)
    b = latent_distances.reshape(_N, _S, _L)
    out = pl.pallas_call(
        _mst_loss_kernel,
        out_shape=jax.ShapeDtypeStruct((1, 1), jnp.float32),
        in_specs=[
            pl.BlockSpec(memory_space=pltpu.VMEM),
            pl.BlockSpec(memory_space=pltpu.VMEM),
        ],
        out_specs=pl.BlockSpec(memory_space=pltpu.SMEM),
        compiler_params=pltpu.CompilerParams(
            vmem_limit_bytes=100 * 1024 * 1024,
        ),
    )(a, b)
    return out[0, 0]
